# baseline (speedup 1.0000x reference)
import functools

import jax
import jax.numpy as jnp
from jax import lax
from jax.experimental import pallas as pl
from jax.experimental.pallas import tpu as pltpu
from jax.experimental.pallas import tpu_sc as plsc

N = 10000
E = 320000
D = 128

NC = 2
NS = 16
NW = NC * NS

CHUNK = 128
G = 2
NPHASE = 5
QP = 16
NGP = QP // G
NCHUNK = NPHASE * QP
EDGES_PT = NCHUNK * CHUNK
E_PAD = EDGES_PT * NW

ROWS_PT = -(-(N + 8) // (NS * 8)) * 8
AGG_ROWS = ROWS_PT * NS

_mesh = plsc.VectorSubcoreMesh(core_axis_name="c", subcore_axis_name="s")


@functools.partial(
    pl.kernel,
    out_type=jax.ShapeDtypeStruct((NC, AGG_ROWS, D), jnp.float32),
    mesh=_mesh,
    scratch_types=[
        pltpu.VMEM((QP, CHUNK), jnp.int32),
        pltpu.VMEM((QP, CHUNK), jnp.int32),
        [pltpu.VMEM((CHUNK, D), jnp.float32) for _ in range(G)],
        [pltpu.SemaphoreType.DMA for _ in range(G)],
        [pltpu.SemaphoreType.DMA for _ in range(G)],
        pltpu.VMEM_SHARED((AGG_ROWS, D), jnp.float32),
    ],
)
def _sc_scatter(x_hbm, row_hbm, col_hbm, zeros_hbm, out_hbm,
                rowv, colv, gath, gsem, ssem, agg):
    cid = lax.axis_index("c")
    sid = lax.axis_index("s")
    wid = cid * NS + sid

    pltpu.sync_copy(zeros_hbm, agg.at[pl.ds(sid * ROWS_PT, ROWS_PT)])
    plsc.subcore_barrier()

    for p in range(NPHASE):
        pltpu.sync_copy(row_hbm.at[wid, pl.ds(p * QP, QP)], rowv)
        pltpu.sync_copy(col_hbm.at[wid, pl.ds(p * QP, QP)], colv)

        for b in range(G):
            pltpu.async_copy(x_hbm.at[colv.at[b]], gath[b], gsem[b])

        def body(g, carry):
            for b in range(G):
                i = g * G + b
                pltpu.make_async_copy(
                    x_hbm.at[colv.at[i]], gath[b], gsem[b]).wait()
                pltpu.async_copy(gath[b], agg.at[rowv.at[i]], ssem[b],
                                 add=True)
            for b in range(G):
                i = g * G + b
                pltpu.make_async_copy(
                    gath[b], agg.at[rowv.at[i]], ssem[b]).wait()
                pltpu.async_copy(x_hbm.at[colv.at[i + G]], gath[b], gsem[b])
            return carry

        lax.fori_loop(0, NGP - 1, body, 0)

        for b in range(G):
            i = (NGP - 1) * G + b
            pltpu.make_async_copy(x_hbm.at[colv.at[i]], gath[b], gsem[b]).wait()
            pltpu.async_copy(gath[b], agg.at[rowv.at[i]], ssem[b], add=True)
        for b in range(G):
            i = (NGP - 1) * G + b
            pltpu.make_async_copy(gath[b], agg.at[rowv.at[i]], ssem[b]).wait()

    plsc.subcore_barrier()
    pltpu.sync_copy(agg.at[pl.ds(sid * ROWS_PT, ROWS_PT)],
                    out_hbm.at[cid, pl.ds(sid * ROWS_PT, ROWS_PT)])


_ROWS_BLK = 1000


def _mlp_body(x_ref, p_ref, w1_ref, b1_ref, w2_ref, b2_ref, o_ref):
    acc = x_ref[...] + p_ref[0] + p_ref[1]
    h = jnp.maximum(
        jnp.dot(acc, w1_ref[...], preferred_element_type=jnp.float32)
        + b1_ref[...], 0.0)
    o_ref[...] = (jnp.dot(h, w2_ref[...], preferred_element_type=jnp.float32)
                  + b2_ref[...])


_mlp = pl.pallas_call(
    _mlp_body,
    out_shape=jax.ShapeDtypeStruct((N, D), jnp.float32),
    grid=(N // _ROWS_BLK,),
    in_specs=[
        pl.BlockSpec((_ROWS_BLK, D), lambda i: (i, 0)),
        pl.BlockSpec((NC, _ROWS_BLK, D), lambda i: (0, i, 0)),
        pl.BlockSpec((D, D), lambda i: (0, 0)),
        pl.BlockSpec((1, D), lambda i: (0, 0)),
        pl.BlockSpec((D, D), lambda i: (0, 0)),
        pl.BlockSpec((1, D), lambda i: (0, 0)),
    ],
    out_specs=pl.BlockSpec((_ROWS_BLK, D), lambda i: (i, 0)),
)


def kernel(x, edge_index, W1, b1, W2, b2):
    row = edge_index[0].astype(jnp.int32)
    col = edge_index[1].astype(jnp.int32)
    npad = E_PAD - E
    pad_iota = jnp.arange(npad, dtype=jnp.int32)
    row_p = jnp.concatenate([row, N + pad_iota % (AGG_ROWS - N)])
    col_p = jnp.concatenate([col, pad_iota % N])
    row_p = row_p.reshape(NW, NCHUNK, CHUNK)
    col_p = col_p.reshape(NW, NCHUNK, CHUNK)
    zeros = jnp.zeros((ROWS_PT, D), jnp.float32)
    partials = _sc_scatter(x, row_p, col_p, zeros)
    return _mlp(x, partials, W1, b1.reshape(1, D), W2, b2.reshape(1, D))


# speedup vs baseline: 1.2191x; 1.2191x over previous
import functools

import jax
import jax.numpy as jnp
from jax import lax
from jax.experimental import pallas as pl
from jax.experimental.pallas import tpu as pltpu
from jax.experimental.pallas import tpu_sc as plsc

N = 10000
E = 320000
D = 128

NC = 2
NS = 16
NW = NC * NS

CHUNK = 128
G = 2
NPHASE = 5
QP = 16
NGP = QP // G
NCHUNK = NPHASE * QP
EDGES_PT = NCHUNK * CHUNK
E_PAD = EDGES_PT * NW

ROWS_PT = -(-(N + 8) // (NS * 8)) * 8
AGG_ROWS = ROWS_PT * NS

_mesh = plsc.VectorSubcoreMesh(core_axis_name="c", subcore_axis_name="s")


@functools.partial(
    pl.kernel,
    out_type=jax.ShapeDtypeStruct((NC, AGG_ROWS, D), jnp.float32),
    mesh=_mesh,
    scratch_types=[
        pltpu.VMEM((QP, CHUNK), jnp.int32),
        pltpu.VMEM((QP, CHUNK), jnp.int32),
        [pltpu.VMEM((CHUNK, D), jnp.float32) for _ in range(G)],
        [pltpu.SemaphoreType.DMA for _ in range(G)],
        [pltpu.SemaphoreType.DMA for _ in range(G)],
        pltpu.VMEM_SHARED((AGG_ROWS, D), jnp.float32),
    ],
)
def _sc_scatter(x_hbm, row_hbm, col_hbm, zeros_hbm, out_hbm,
                rowv, colv, gath, gsem, ssem, agg):
    cid = lax.axis_index("c")
    sid = lax.axis_index("s")
    wid = cid * NS + sid

    pltpu.sync_copy(zeros_hbm, agg.at[pl.ds(sid * ROWS_PT, ROWS_PT)])
    plsc.subcore_barrier()

    def phase(p, carry):
        pltpu.sync_copy(row_hbm.at[wid, pl.ds(p * QP, QP)], rowv)
        pltpu.sync_copy(col_hbm.at[wid, pl.ds(p * QP, QP)], colv)

        for b in range(G):
            pltpu.async_copy(x_hbm.at[colv.at[b]], gath[b], gsem[b])

        for g in range(NGP - 1):
            for b in range(G):
                i = g * G + b
                pltpu.make_async_copy(
                    x_hbm.at[colv.at[i]], gath[b], gsem[b]).wait()
                pltpu.async_copy(
                    gath[b], agg.at[rowv.at[i]], ssem[b], add=True).wait()
                pltpu.async_copy(x_hbm.at[colv.at[i + G]], gath[b], gsem[b])

        for b in range(G):
            i = (NGP - 1) * G + b
            pltpu.make_async_copy(x_hbm.at[colv.at[i]], gath[b], gsem[b]).wait()
            pltpu.async_copy(
                gath[b], agg.at[rowv.at[i]], ssem[b], add=True).wait()
        return carry

    lax.fori_loop(0, NPHASE, phase, 0)

    plsc.subcore_barrier()
    pltpu.sync_copy(agg.at[pl.ds(sid * ROWS_PT, ROWS_PT)],
                    out_hbm.at[cid, pl.ds(sid * ROWS_PT, ROWS_PT)])


_ROWS_BLK = 1000


def _mlp_body(x_ref, p_ref, w1_ref, b1_ref, w2_ref, b2_ref, o_ref):
    acc = x_ref[...] + p_ref[0] + p_ref[1]
    h = jnp.maximum(
        jnp.dot(acc, w1_ref[...], preferred_element_type=jnp.float32)
        + b1_ref[...], 0.0)
    o_ref[...] = (jnp.dot(h, w2_ref[...], preferred_element_type=jnp.float32)
                  + b2_ref[...])


_mlp = pl.pallas_call(
    _mlp_body,
    out_shape=jax.ShapeDtypeStruct((N, D), jnp.float32),
    grid=(N // _ROWS_BLK,),
    in_specs=[
        pl.BlockSpec((_ROWS_BLK, D), lambda i: (i, 0)),
        pl.BlockSpec((NC, _ROWS_BLK, D), lambda i: (0, i, 0)),
        pl.BlockSpec((D, D), lambda i: (0, 0)),
        pl.BlockSpec((1, D), lambda i: (0, 0)),
        pl.BlockSpec((D, D), lambda i: (0, 0)),
        pl.BlockSpec((1, D), lambda i: (0, 0)),
    ],
    out_specs=pl.BlockSpec((_ROWS_BLK, D), lambda i: (i, 0)),
)


def kernel(x, edge_index, W1, b1, W2, b2):
    row = edge_index[0].astype(jnp.int32)
    col = edge_index[1].astype(jnp.int32)
    npad = E_PAD - E
    pad_iota = jnp.arange(npad, dtype=jnp.int32)
    row_p = jnp.concatenate([row, N + pad_iota % (AGG_ROWS - N)])
    col_p = jnp.concatenate([col, pad_iota % N])
    row_p = row_p.reshape(NW, NCHUNK, CHUNK)
    col_p = col_p.reshape(NW, NCHUNK, CHUNK)
    zeros = jnp.zeros((ROWS_PT, D), jnp.float32)
    partials = _sc_scatter(x, row_p, col_p, zeros)
    return _mlp(x, partials, W1, b1.reshape(1, D), W2, b2.reshape(1, D))


# speedup vs baseline: 1.2931x; 1.0607x over previous
import functools

import jax
import jax.numpy as jnp
from jax import lax
from jax.experimental import pallas as pl
from jax.experimental.pallas import tpu as pltpu
from jax.experimental.pallas import tpu_sc as plsc

N = 10000
E = 320000
D = 128

NC = 2
NS = 16
NW = NC * NS

CHUNK = 128
G = 2
QP = 8
NPAIR = 5
NCHUNK = NPAIR * 2 * QP
PAIRC = 2 * QP
EDGES_PT = NCHUNK * CHUNK
E_PAD = EDGES_PT * NW

ROWS_PT = -(-(N + 8) // (NS * 8)) * 8
AGG_ROWS = ROWS_PT * NS

_mesh = plsc.VectorSubcoreMesh(core_axis_name="c", subcore_axis_name="s")


@functools.partial(
    pl.kernel,
    out_type=jax.ShapeDtypeStruct((NC, AGG_ROWS, D), jnp.float32),
    mesh=_mesh,
    scratch_types=[
        [pltpu.VMEM((QP, CHUNK), jnp.int32) for _ in range(2)],
        [pltpu.VMEM((QP, CHUNK), jnp.int32) for _ in range(2)],
        [pltpu.VMEM((CHUNK, D), jnp.float32) for _ in range(G)],
        [pltpu.SemaphoreType.DMA for _ in range(G)],
        [pltpu.SemaphoreType.DMA for _ in range(G)],
        [pltpu.SemaphoreType.DMA for _ in range(2)],
        pltpu.VMEM_SHARED((AGG_ROWS, D), jnp.float32),
    ],
)
def _sc_scatter(x_hbm, row_hbm, col_hbm, out_hbm,
                rowv, colv, gath, gsem, ssem, isem, agg):
    cid = lax.axis_index("c")
    sid = lax.axis_index("s")
    wid = cid * NS + sid

    def zrow(j, carry):
        for k in range(D // 16):
            gath[0][j, pl.ds(k * 16, 16)] = jnp.zeros((16,), jnp.float32)
        return carry

    lax.fori_loop(0, CHUNK, zrow, 0)
    base_row = sid * ROWS_PT
    for c in range(ROWS_PT // CHUNK):
        pltpu.sync_copy(gath[0], agg.at[pl.ds(base_row + c * CHUNK, CHUNK)])
    _rem = ROWS_PT % CHUNK
    if _rem:
        pltpu.sync_copy(gath[0].at[pl.ds(0, _rem)],
                        agg.at[pl.ds(base_row + (ROWS_PT // CHUNK) * CHUNK,
                                     _rem)])
    plsc.subcore_barrier()

    def idx_load(pair, half, buf):
        off = jnp.minimum((pair * 2 + half) * QP, NCHUNK - QP)
        pltpu.async_copy(row_hbm.at[wid, pl.ds(off, QP)], rowv[buf], isem[buf])
        pltpu.async_copy(col_hbm.at[wid, pl.ds(off, QP)], colv[buf], isem[buf])

    def idx_wait(buf):
        pltpu.make_async_copy(row_hbm.at[wid, pl.ds(0, QP)], rowv[buf],
                              isem[buf]).wait()
        pltpu.make_async_copy(col_hbm.at[wid, pl.ds(0, QP)], colv[buf],
                              isem[buf]).wait()

    idx_load(0, 0, 0)

    def pair_body(pair, carry):
        idx_load(pair, 1, 1)
        idx_wait(0)

        def cv(i):
            return colv[i // QP].at[i % QP]

        def rv(i):
            return rowv[i // QP].at[i % QP]

        for b in range(G):
            pltpu.async_copy(x_hbm.at[cv(b)], gath[b], gsem[b])

        for i in range(PAIRC - G):
            b = i % G
            pltpu.make_async_copy(x_hbm.at[cv(i)], gath[b], gsem[b]).wait()
            pltpu.async_copy(gath[b], agg.at[rv(i)], ssem[b], add=True).wait()
            nxt = i + G
            if nxt == QP:
                idx_wait(1)
            if nxt == QP + 2:
                idx_load(pair + 1, 0, 0)
            pltpu.async_copy(x_hbm.at[cv(nxt)], gath[b], gsem[b])

        for i in range(PAIRC - G, PAIRC):
            b = i % G
            pltpu.make_async_copy(x_hbm.at[cv(i)], gath[b], gsem[b]).wait()
            pltpu.async_copy(gath[b], agg.at[rv(i)], ssem[b], add=True).wait()
        return carry

    lax.fori_loop(0, NPAIR, pair_body, 0)

    idx_wait(0)

    plsc.subcore_barrier()
    pltpu.sync_copy(agg.at[pl.ds(base_row, ROWS_PT)],
                    out_hbm.at[cid, pl.ds(base_row, ROWS_PT)])


_ROWS_BLK = 1000


def _mlp_body(x_ref, p_ref, w1_ref, b1_ref, w2_ref, b2_ref, o_ref):
    acc = x_ref[...] + p_ref[0] + p_ref[1]
    h = jnp.maximum(
        jnp.dot(acc, w1_ref[...], preferred_element_type=jnp.float32)
        + b1_ref[...], 0.0)
    o_ref[...] = (jnp.dot(h, w2_ref[...], preferred_element_type=jnp.float32)
                  + b2_ref[...])


_mlp = pl.pallas_call(
    _mlp_body,
    out_shape=jax.ShapeDtypeStruct((N, D), jnp.float32),
    grid=(N // _ROWS_BLK,),
    in_specs=[
        pl.BlockSpec((_ROWS_BLK, D), lambda i: (i, 0)),
        pl.BlockSpec((NC, _ROWS_BLK, D), lambda i: (0, i, 0)),
        pl.BlockSpec((D, D), lambda i: (0, 0)),
        pl.BlockSpec((1, D), lambda i: (0, 0)),
        pl.BlockSpec((D, D), lambda i: (0, 0)),
        pl.BlockSpec((1, D), lambda i: (0, 0)),
    ],
    out_specs=pl.BlockSpec((_ROWS_BLK, D), lambda i: (i, 0)),
)


def kernel(x, edge_index, W1, b1, W2, b2):
    row = edge_index[0].astype(jnp.int32)
    col = edge_index[1].astype(jnp.int32)
    npad = E_PAD - E
    pad_iota = jnp.arange(npad, dtype=jnp.int32)
    row_p = jnp.concatenate([row, N + pad_iota % (AGG_ROWS - N)])
    col_p = jnp.concatenate([col, pad_iota % N])
    row_p = row_p.reshape(NW, NCHUNK, CHUNK)
    col_p = col_p.reshape(NW, NCHUNK, CHUNK)
    partials = _sc_scatter(x, row_p, col_p)
    return _mlp(x, partials, W1, b1.reshape(1, D), W2, b2.reshape(1, D))


# speedup vs baseline: 1.3271x; 1.0262x over previous
import functools

import jax
import jax.numpy as jnp
from jax import lax
from jax.experimental import pallas as pl
from jax.experimental.pallas import tpu as pltpu
from jax.experimental.pallas import tpu_sc as plsc

N = 10000
E = 320000
D = 128

NC = 2
NS = 16
NW = NC * NS

CHUNK = 128
G = 2
QP = 8
NPAIR = 5
NCHUNK = NPAIR * 2 * QP
PAIRC = 2 * QP
EDGES_PT = NCHUNK * CHUNK
E_PAD = EDGES_PT * NW

ROWS_PT = -(-(N + 8) // (NS * 8)) * 8
AGG_ROWS = ROWS_PT * NS

_mesh = plsc.VectorSubcoreMesh(core_axis_name="c", subcore_axis_name="s")


@functools.partial(
    pl.kernel,
    out_type=jax.ShapeDtypeStruct((NC, AGG_ROWS, D), jnp.float32),
    mesh=_mesh,
    scratch_types=[
        [pltpu.VMEM((QP, CHUNK), jnp.int32) for _ in range(2)],
        [pltpu.VMEM((QP, CHUNK), jnp.int32) for _ in range(2)],
        [pltpu.VMEM((CHUNK, D), jnp.float32) for _ in range(G)],
        [pltpu.SemaphoreType.DMA for _ in range(G)],
        [pltpu.SemaphoreType.DMA for _ in range(G)],
        [pltpu.SemaphoreType.DMA for _ in range(2)],
        pltpu.VMEM_SHARED((AGG_ROWS, D), jnp.float32),
    ],
)
def _sc_scatter(x_hbm, row_hbm, col_hbm, out_hbm,
                rowv, colv, gath, gsem, ssem, isem, agg):
    cid = lax.axis_index("c")
    sid = lax.axis_index("s")
    wid = cid * NS + sid

    def zrow(j, carry):
        for k in range(D // 16):
            gath[0][j, pl.ds(k * 16, 16)] = jnp.zeros((16,), jnp.float32)
        return carry

    lax.fori_loop(0, CHUNK, zrow, 0)
    base_row = sid * ROWS_PT
    for c in range(ROWS_PT // CHUNK):
        pltpu.sync_copy(gath[0], agg.at[pl.ds(base_row + c * CHUNK, CHUNK)])
    _rem = ROWS_PT % CHUNK
    if _rem:
        pltpu.sync_copy(gath[0].at[pl.ds(0, _rem)],
                        agg.at[pl.ds(base_row + (ROWS_PT // CHUNK) * CHUNK,
                                     _rem)])
    plsc.subcore_barrier()

    def idx_load(pair, half, buf):
        off = jnp.minimum((pair * 2 + half) * QP, NCHUNK - QP)
        pltpu.async_copy(row_hbm.at[wid, pl.ds(off, QP)], rowv[buf], isem[buf])
        pltpu.async_copy(col_hbm.at[wid, pl.ds(off, QP)], colv[buf], isem[buf])

    def idx_wait(buf):
        pltpu.make_async_copy(row_hbm.at[wid, pl.ds(0, QP)], rowv[buf],
                              isem[buf]).wait()
        pltpu.make_async_copy(col_hbm.at[wid, pl.ds(0, QP)], colv[buf],
                              isem[buf]).wait()

    def cv(i):
        return colv[(i // QP) % 2].at[i % QP]

    def rv(i):
        return rowv[(i // QP) % 2].at[i % QP]

    idx_load(0, 0, 0)
    idx_load(0, 1, 1)
    idx_wait(0)
    for b in range(G):
        pltpu.async_copy(x_hbm.at[cv(b)], gath[b], gsem[b])

    def pair_body(pair, carry):
        for i in range(PAIRC):
            b = i % G
            pltpu.make_async_copy(x_hbm.at[cv(i)], gath[b], gsem[b]).wait()
            pltpu.async_copy(gath[b], agg.at[rv(i)], ssem[b], add=True).wait()
            nxt = i + G
            if nxt == QP:
                idx_wait(1)
            if nxt == QP + 2:
                idx_load(pair + 1, 0, 0)
            if i == PAIRC - 1:
                idx_load(pair + 1, 1, 1)
            if nxt >= PAIRC:
                if nxt == PAIRC:
                    idx_wait(0)
                pltpu.async_copy(x_hbm.at[cv(nxt - PAIRC)], gath[b], gsem[b])
            else:
                pltpu.async_copy(x_hbm.at[cv(nxt)], gath[b], gsem[b])
        return carry

    lax.fori_loop(0, NPAIR, pair_body, 0)

    for b in range(G):
        pltpu.make_async_copy(x_hbm.at[cv(b)], gath[b], gsem[b]).wait()
    idx_wait(1)

    plsc.subcore_barrier()
    pltpu.sync_copy(agg.at[pl.ds(base_row, ROWS_PT)],
                    out_hbm.at[cid, pl.ds(base_row, ROWS_PT)])


_ROWS_BLK = 1000


def _mlp_body(x_ref, p_ref, w1_ref, b1_ref, w2_ref, b2_ref, o_ref):
    acc = x_ref[...] + p_ref[0] + p_ref[1]
    h = jnp.maximum(
        jnp.dot(acc, w1_ref[...], preferred_element_type=jnp.float32)
        + b1_ref[...], 0.0)
    o_ref[...] = (jnp.dot(h, w2_ref[...], preferred_element_type=jnp.float32)
                  + b2_ref[...])


_mlp = pl.pallas_call(
    _mlp_body,
    out_shape=jax.ShapeDtypeStruct((N, D), jnp.float32),
    grid=(N // _ROWS_BLK,),
    in_specs=[
        pl.BlockSpec((_ROWS_BLK, D), lambda i: (i, 0)),
        pl.BlockSpec((NC, _ROWS_BLK, D), lambda i: (0, i, 0)),
        pl.BlockSpec((D, D), lambda i: (0, 0)),
        pl.BlockSpec((1, D), lambda i: (0, 0)),
        pl.BlockSpec((D, D), lambda i: (0, 0)),
        pl.BlockSpec((1, D), lambda i: (0, 0)),
    ],
    out_specs=pl.BlockSpec((_ROWS_BLK, D), lambda i: (i, 0)),
)


def kernel(x, edge_index, W1, b1, W2, b2):
    row = edge_index[0].astype(jnp.int32)
    col = edge_index[1].astype(jnp.int32)
    npad = E_PAD - E
    pad_iota = jnp.arange(npad, dtype=jnp.int32)
    row_p = jnp.concatenate([row, N + pad_iota % (AGG_ROWS - N)])
    col_p = jnp.concatenate([col, pad_iota % N])
    row_p = row_p.reshape(NW, NCHUNK, CHUNK)
    col_p = col_p.reshape(NW, NCHUNK, CHUNK)
    partials = _sc_scatter(x, row_p, col_p)
    return _mlp(x, partials, W1, b1.reshape(1, D), W2, b2.reshape(1, D))


# speedup vs baseline: 1.3310x; 1.0030x over previous
import functools

import jax
import jax.numpy as jnp
from jax import lax
from jax.experimental import pallas as pl
from jax.experimental.pallas import tpu as pltpu
from jax.experimental.pallas import tpu_sc as plsc

N = 10000
E = 320000
D = 128

NC = 2
NS = 16
NW = NC * NS

CHUNK = 128
G = 2
QP = 8
NPAIR = 5
NCHUNK = NPAIR * 2 * QP
PAIRC = 2 * QP
EDGES_PT = NCHUNK * CHUNK
E_PAD = EDGES_PT * NW

ROWS_PT = -(-(N + 8) // (NS * 8)) * 8
AGG_ROWS = ROWS_PT * NS

_mesh = plsc.VectorSubcoreMesh(core_axis_name="c", subcore_axis_name="s")


@functools.partial(
    pl.kernel,
    out_type=jax.ShapeDtypeStruct((NC, AGG_ROWS, D), jnp.float32),
    mesh=_mesh,
    scratch_types=[
        [pltpu.VMEM((QP, CHUNK), jnp.int32) for _ in range(2)],
        [pltpu.VMEM((QP, CHUNK), jnp.int32) for _ in range(2)],
        [pltpu.VMEM((CHUNK, D), jnp.float32) for _ in range(G)],
        [pltpu.SemaphoreType.DMA for _ in range(G)],
        [pltpu.SemaphoreType.DMA for _ in range(G)],
        [pltpu.SemaphoreType.DMA for _ in range(2)],
        pltpu.VMEM_SHARED((AGG_ROWS, D), jnp.float32),
    ],
)
def _sc_scatter(x_hbm, row_hbm, col_hbm, out_hbm,
                rowv, colv, gath, gsem, ssem, isem, agg):
    cid = lax.axis_index("c")
    sid = lax.axis_index("s")
    wid = cid * NS + sid

    def zrow(j, carry):
        for k in range(D // 16):
            gath[0][j, pl.ds(k * 16, 16)] = jnp.zeros((16,), jnp.float32)
        return carry

    lax.fori_loop(0, CHUNK, zrow, 0)
    base_row = sid * ROWS_PT
    for c in range(ROWS_PT // CHUNK):
        pltpu.sync_copy(gath[0], agg.at[pl.ds(base_row + c * CHUNK, CHUNK)])
    _rem = ROWS_PT % CHUNK
    if _rem:
        pltpu.sync_copy(gath[0].at[pl.ds(0, _rem)],
                        agg.at[pl.ds(base_row + (ROWS_PT // CHUNK) * CHUNK,
                                     _rem)])
    plsc.subcore_barrier()

    def idx_load(pair, half, buf):
        off = jnp.minimum((pair * 2 + half) * QP, NCHUNK - QP)
        pltpu.async_copy(row_hbm.at[wid, pl.ds(off, QP)], rowv[buf], isem[buf])
        pltpu.async_copy(col_hbm.at[wid, pl.ds(off, QP)], colv[buf], isem[buf])

    def idx_wait(buf):
        pltpu.make_async_copy(row_hbm.at[wid, pl.ds(0, QP)], rowv[buf],
                              isem[buf]).wait()
        pltpu.make_async_copy(col_hbm.at[wid, pl.ds(0, QP)], colv[buf],
                              isem[buf]).wait()

    def cv(i):
        return colv[(i // QP) % 2].at[i % QP]

    def rv(i):
        return rowv[(i // QP) % 2].at[i % QP]

    idx_load(0, 0, 0)
    idx_load(0, 1, 1)
    idx_wait(0)
    for b in range(G):
        pltpu.async_copy(x_hbm.at[cv(b)], gath[b], gsem[b])

    def gwait(b):
        pltpu.make_async_copy(x_hbm.at[pl.ds(0, CHUNK)], gath[b],
                              gsem[b]).wait()

    def pair_body(pair, carry):
        for i in range(PAIRC):
            b = i % G
            gwait(b)
            pltpu.async_copy(gath[b], agg.at[rv(i)], ssem[b], add=True).wait()
            nxt = i + G
            if nxt == QP:
                idx_wait(1)
            if nxt == QP + 2:
                idx_load(pair + 1, 0, 0)
            if i == PAIRC - 1:
                idx_load(pair + 1, 1, 1)
            if nxt >= PAIRC:
                if nxt == PAIRC:
                    idx_wait(0)
                pltpu.async_copy(x_hbm.at[cv(nxt - PAIRC)], gath[b], gsem[b])
            else:
                pltpu.async_copy(x_hbm.at[cv(nxt)], gath[b], gsem[b])
        return carry

    lax.fori_loop(0, NPAIR, pair_body, 0)

    for b in range(G):
        gwait(b)
    idx_wait(1)

    plsc.subcore_barrier()
    pltpu.sync_copy(agg.at[pl.ds(base_row, ROWS_PT)],
                    out_hbm.at[cid, pl.ds(base_row, ROWS_PT)])


_ROWS_BLK = 1000


def _mlp_body(x_ref, p_ref, w1_ref, b1_ref, w2_ref, b2_ref, o_ref):
    acc = x_ref[...] + p_ref[0] + p_ref[1]
    h = jnp.maximum(
        jnp.dot(acc, w1_ref[...], preferred_element_type=jnp.float32)
        + b1_ref[...], 0.0)
    o_ref[...] = (jnp.dot(h, w2_ref[...], preferred_element_type=jnp.float32)
                  + b2_ref[...])


_mlp = pl.pallas_call(
    _mlp_body,
    out_shape=jax.ShapeDtypeStruct((N, D), jnp.float32),
    grid=(N // _ROWS_BLK,),
    in_specs=[
        pl.BlockSpec((_ROWS_BLK, D), lambda i: (i, 0)),
        pl.BlockSpec((NC, _ROWS_BLK, D), lambda i: (0, i, 0)),
        pl.BlockSpec((D, D), lambda i: (0, 0)),
        pl.BlockSpec((1, D), lambda i: (0, 0)),
        pl.BlockSpec((D, D), lambda i: (0, 0)),
        pl.BlockSpec((1, D), lambda i: (0, 0)),
    ],
    out_specs=pl.BlockSpec((_ROWS_BLK, D), lambda i: (i, 0)),
)


def kernel(x, edge_index, W1, b1, W2, b2):
    row = edge_index[0].astype(jnp.int32)
    col = edge_index[1].astype(jnp.int32)
    npad = E_PAD - E
    pad_iota = jnp.arange(npad, dtype=jnp.int32)
    row_p = jnp.concatenate([row, N + pad_iota % (AGG_ROWS - N)])
    col_p = jnp.concatenate([col, pad_iota % N])
    row_p = row_p.reshape(NW, NCHUNK, CHUNK)
    col_p = col_p.reshape(NW, NCHUNK, CHUNK)
    partials = _sc_scatter(x, row_p, col_p)
    return _mlp(x, partials, W1, b1.reshape(1, D), W2, b2.reshape(1, D))


# speedup vs baseline: 1.4461x; 1.0864x over previous
import functools

import jax
import jax.numpy as jnp
from jax import lax
from jax.experimental import pallas as pl
from jax.experimental.pallas import tpu as pltpu
from jax.experimental.pallas import tpu_sc as plsc

N = 10000
E = 320000
D = 128

NC = 2
NS = 16
NW = NC * NS

CHUNK = 64
G = 4
QP = 8
NPAIR = 10
NCHUNK = NPAIR * 2 * QP
PAIRC = 2 * QP
EDGES_PT = NCHUNK * CHUNK
E_PAD = EDGES_PT * NW

ROWS_PT = -(-(N + 8) // (NS * 8)) * 8
AGG_ROWS = ROWS_PT * NS

_mesh = plsc.VectorSubcoreMesh(core_axis_name="c", subcore_axis_name="s")


@functools.partial(
    pl.kernel,
    out_type=jax.ShapeDtypeStruct((NC, AGG_ROWS, D), jnp.float32),
    mesh=_mesh,
    scratch_types=[
        [pltpu.VMEM((QP, CHUNK), jnp.int32) for _ in range(2)],
        [pltpu.VMEM((QP, CHUNK), jnp.int32) for _ in range(2)],
        [pltpu.VMEM((CHUNK, D), jnp.float32) for _ in range(G)],
        [pltpu.SemaphoreType.DMA for _ in range(G)],
        [pltpu.SemaphoreType.DMA for _ in range(G)],
        [pltpu.SemaphoreType.DMA for _ in range(2)],
        pltpu.VMEM_SHARED((AGG_ROWS, D), jnp.float32),
    ],
)
def _sc_scatter(x_hbm, row_hbm, col_hbm, out_hbm,
                rowv, colv, gath, gsem, ssem, isem, agg):
    cid = lax.axis_index("c")
    sid = lax.axis_index("s")
    wid = cid * NS + sid

    def zrow(j, carry):
        for k in range(D // 16):
            gath[0][j, pl.ds(k * 16, 16)] = jnp.zeros((16,), jnp.float32)
        return carry

    lax.fori_loop(0, CHUNK, zrow, 0)
    base_row = sid * ROWS_PT
    for c in range(ROWS_PT // CHUNK):
        pltpu.sync_copy(gath[0], agg.at[pl.ds(base_row + c * CHUNK, CHUNK)])
    _rem = ROWS_PT % CHUNK
    if _rem:
        pltpu.sync_copy(gath[0].at[pl.ds(0, _rem)],
                        agg.at[pl.ds(base_row + (ROWS_PT // CHUNK) * CHUNK,
                                     _rem)])
    plsc.subcore_barrier()

    def idx_load(pair, half, buf):
        off = jnp.minimum((pair * 2 + half) * QP, NCHUNK - QP)
        pltpu.async_copy(row_hbm.at[wid, pl.ds(off, QP)], rowv[buf], isem[buf])
        pltpu.async_copy(col_hbm.at[wid, pl.ds(off, QP)], colv[buf], isem[buf])

    def idx_wait(buf):
        pltpu.make_async_copy(row_hbm.at[wid, pl.ds(0, QP)], rowv[buf],
                              isem[buf]).wait()
        pltpu.make_async_copy(col_hbm.at[wid, pl.ds(0, QP)], colv[buf],
                              isem[buf]).wait()

    def cv(i):
        return colv[(i // QP) % 2].at[i % QP]

    def rv(i):
        return rowv[(i // QP) % 2].at[i % QP]

    idx_load(0, 0, 0)
    idx_load(0, 1, 1)
    idx_wait(0)
    for b in range(G):
        pltpu.async_copy(x_hbm.at[cv(b)], gath[b], gsem[b])

    def gwait(b):
        pltpu.make_async_copy(x_hbm.at[pl.ds(0, CHUNK)], gath[b],
                              gsem[b]).wait()

    def pair_body(pair, carry):
        for i in range(PAIRC):
            b = i % G
            gwait(b)
            pltpu.async_copy(gath[b], agg.at[rv(i)], ssem[b], add=True).wait()
            nxt = i + G
            if nxt == QP:
                idx_wait(1)
            if i == QP - 1:
                idx_load(pair + 1, 0, 0)
            if i == PAIRC - 1:
                idx_load(pair + 1, 1, 1)
            if nxt >= PAIRC:
                if nxt == PAIRC:
                    idx_wait(0)
                pltpu.async_copy(x_hbm.at[cv(nxt - PAIRC)], gath[b], gsem[b])
            else:
                pltpu.async_copy(x_hbm.at[cv(nxt)], gath[b], gsem[b])
        return carry

    lax.fori_loop(0, NPAIR, pair_body, 0)

    for b in range(G):
        gwait(b)
    idx_wait(1)

    plsc.subcore_barrier()
    pltpu.sync_copy(agg.at[pl.ds(base_row, ROWS_PT)],
                    out_hbm.at[cid, pl.ds(base_row, ROWS_PT)])


_ROWS_BLK = 1000


def _mlp_body(x_ref, p_ref, w1_ref, b1_ref, w2_ref, b2_ref, o_ref):
    acc = x_ref[...] + p_ref[0] + p_ref[1]
    h = jnp.maximum(
        jnp.dot(acc, w1_ref[...], preferred_element_type=jnp.float32)
        + b1_ref[...], 0.0)
    o_ref[...] = (jnp.dot(h, w2_ref[...], preferred_element_type=jnp.float32)
                  + b2_ref[...])


_mlp = pl.pallas_call(
    _mlp_body,
    out_shape=jax.ShapeDtypeStruct((N, D), jnp.float32),
    grid=(N // _ROWS_BLK,),
    in_specs=[
        pl.BlockSpec((_ROWS_BLK, D), lambda i: (i, 0)),
        pl.BlockSpec((NC, _ROWS_BLK, D), lambda i: (0, i, 0)),
        pl.BlockSpec((D, D), lambda i: (0, 0)),
        pl.BlockSpec((1, D), lambda i: (0, 0)),
        pl.BlockSpec((D, D), lambda i: (0, 0)),
        pl.BlockSpec((1, D), lambda i: (0, 0)),
    ],
    out_specs=pl.BlockSpec((_ROWS_BLK, D), lambda i: (i, 0)),
)


def kernel(x, edge_index, W1, b1, W2, b2):
    row = edge_index[0].astype(jnp.int32)
    col = edge_index[1].astype(jnp.int32)
    npad = E_PAD - E
    pad_iota = jnp.arange(npad, dtype=jnp.int32)
    row_p = jnp.concatenate([row, N + pad_iota % (AGG_ROWS - N)])
    col_p = jnp.concatenate([col, pad_iota % N])
    row_p = row_p.reshape(NW, NCHUNK, CHUNK)
    col_p = col_p.reshape(NW, NCHUNK, CHUNK)
    partials = _sc_scatter(x, row_p, col_p)
    return _mlp(x, partials, W1, b1.reshape(1, D), W2, b2.reshape(1, D))
